# baseline (speedup 1.0000x reference)
import functools

import jax
import jax.numpy as jnp
from jax import lax
from jax.experimental import pallas as pl
from jax.experimental.pallas import tpu as pltpu
from jax.experimental.pallas import tpu_sc as plsc

T = 2048
H = 1024
I = 1024
E = 8
K = 2
P = T * K
TILE = 256
NT = 24
NP = NT * TILE

NC, NS = 2, 16
NW = NC * NS
RPW = NP // NW
GCH = 64
TPW = T // NW
CCH = 16


def _router_kernel(x_ref, gw_ref, dest_ref, w_ref, xi_ref, meta_ref):
    xi_ref[...] = x_ref[...].astype(jnp.bfloat16)
    logits = lax.dot_general(
        x_ref[...], gw_ref[...], (((1,), (1,)), ((), ())),
        preferred_element_type=jnp.float32)
    a1 = jnp.argmax(logits, axis=1)
    l1 = jnp.max(logits, axis=1)
    cols = lax.broadcasted_iota(jnp.int32, (T, E), 1)
    masked = jnp.where(cols == a1[:, None], -jnp.inf, logits)
    a2 = jnp.argmax(masked, axis=1)
    l2 = jnp.max(masked, axis=1)
    w1 = jax.nn.sigmoid(l1 - l2)
    w_ref[0, :] = w1
    w_ref[1, :] = 1.0 - w1

    onea = (cols == a1[:, None]).astype(jnp.float32)
    oneb = (cols == a2[:, None]).astype(jnp.float32)
    s = onea + oneb
    cum = s
    for step in (1, 2, 4, 8, 16, 32, 64, 128, 256, 512, 1024):
        cum = cum + jnp.concatenate(
            [jnp.zeros((step, E), jnp.float32), cum[:T - step]], axis=0)
    cum_excl = cum - s
    counts = cum[T - 1, :].reshape(1, E)
    pad_counts = jnp.floor((counts + (TILE - 1)) * (1.0 / TILE)) * TILE
    rr_r = lax.broadcasted_iota(jnp.int32, (E, E), 0)
    rr_c = lax.broadcasted_iota(jnp.int32, (E, E), 1)
    tri_x = (rr_c > rr_r).astype(jnp.float32)
    pad_off = jnp.dot(pad_counts, tri_x,
                      preferred_element_type=jnp.float32)
    rank0 = jnp.sum(onea * cum_excl, axis=1)
    rank1 = jnp.sum(oneb * cum_excl, axis=1)
    off0 = jnp.sum(onea * pad_off, axis=1)
    off1 = jnp.sum(oneb * pad_off, axis=1)
    dest_ref[0, :] = (off0 + rank0).astype(jnp.int32)
    dest_ref[1, :] = (off1 + rank1).astype(jnp.int32)

    pad_end = pad_off + pad_counts
    n_tiles_f = pad_end[0, E - 1] * (1.0 / TILE)
    i128 = lax.broadcasted_iota(jnp.int32, (128,), 0).astype(jnp.float32)
    g128 = i128 * float(TILE)
    te = jnp.minimum(
        jnp.sum((g128[:, None] >= pad_end).astype(jnp.float32), axis=1),
        float(E - 1))
    last_e = jnp.sum(jnp.where(i128 == n_tiles_f - 1.0, te, 0.0))
    te = jnp.where(i128 < n_tiles_f, te, last_e)
    meta = jnp.where(i128 == float(NT), n_tiles_f, te)
    meta_ref[0, :] = meta.astype(jnp.int32)


def _router(x, gate_proj_w):
    return pl.pallas_call(
        _router_kernel,
        in_specs=[
            pl.BlockSpec((T, H), lambda: (0, 0)),
            pl.BlockSpec((E, H), lambda: (0, 0)),
        ],
        out_specs=[
            pl.BlockSpec((K, T), lambda: (0, 0)),
            pl.BlockSpec((K, T), lambda: (0, 0)),
            pl.BlockSpec((T, H), lambda: (0, 0)),
            pl.BlockSpec((1, 128), lambda: (0, 0)),
        ],
        out_shape=[
            jax.ShapeDtypeStruct((K, T), jnp.int32),
            jax.ShapeDtypeStruct((K, T), jnp.float32),
            jax.ShapeDtypeStruct((T, H), jnp.bfloat16),
            jax.ShapeDtypeStruct((1, 128), jnp.int32),
        ],
    )(x, gate_proj_w)


JI = 2


def _ffn_kernel(meta_ref, xb_ref, dest_ref, w2_ref, wg_ref, wu_ref, wd_ref,
                ys_ref, xg_s, w_s):
    g = pl.program_id(0)
    j = pl.program_id(1)

    @pl.when(g < meta_ref[NT])
    def _():
        @pl.when(j == 0)
        def _gather():
            rows = lax.broadcasted_iota(jnp.int32, (TILE, T), 0) + g * TILE
            m0 = (dest_ref[0, :][None, :] == rows).astype(jnp.bfloat16)
            m1 = (dest_ref[1, :][None, :] == rows).astype(jnp.bfloat16)
            ms = m0 + m1
            xg_s[...] = jnp.dot(
                ms, xb_ref[...],
                preferred_element_type=jnp.float32).astype(jnp.bfloat16)
            w_s[...] = (
                jnp.dot(m0, w2_ref[0, :].astype(jnp.bfloat16),
                        preferred_element_type=jnp.float32)
                + jnp.dot(m1, w2_ref[1, :].astype(jnp.bfloat16),
                          preferred_element_type=jnp.float32))[:, None]

        xg = xg_s[...]
        wg = wg_ref[0].astype(jnp.bfloat16)
        wu = wu_ref[0].astype(jnp.bfloat16)
        wd = wd_ref[0].astype(jnp.bfloat16)
        gate = jnp.dot(xg, wg, preferred_element_type=jnp.float32)
        up = jnp.dot(xg, wu, preferred_element_type=jnp.float32)
        inter = (gate * jax.nn.sigmoid(gate) * up).astype(jnp.bfloat16)
        d = jnp.dot(inter, wd, preferred_element_type=jnp.float32)
        contrib = w_s[...] * d

        @pl.when(j == 0)
        def _init():
            ys_ref[...] = contrib

        @pl.when(j != 0)
        def _acc():
            ys_ref[...] += contrib


def _ffn(meta, xb, dests, w2, gate_weights, up_weights, down_weights):
    grid_spec = pltpu.PrefetchScalarGridSpec(
        num_scalar_prefetch=1,
        grid=(NT, JI),
        in_specs=[
            pl.BlockSpec((T, H), lambda g, j, m: (0, 0)),
            pl.BlockSpec((K, T), lambda g, j, m: (0, 0)),
            pl.BlockSpec((K, T), lambda g, j, m: (0, 0)),
            pl.BlockSpec((1, H, I // JI), lambda g, j, m: (m[g], 0, j)),
            pl.BlockSpec((1, H, I // JI), lambda g, j, m: (m[g], 0, j)),
            pl.BlockSpec((1, I // JI, H), lambda g, j, m: (m[g], j, 0)),
        ],
        out_specs=pl.BlockSpec((TILE, H), lambda g, j, m: (g, 0)),
        scratch_shapes=[
            pltpu.VMEM((TILE, H), jnp.bfloat16),
            pltpu.VMEM((TILE, 1), jnp.float32),
        ],
    )
    return pl.pallas_call(
        _ffn_kernel,
        grid_spec=grid_spec,
        out_shape=jax.ShapeDtypeStruct((NP, H), jnp.float32),
    )(meta, xb, dests, w2, gate_weights, up_weights, down_weights)


CNCH = TPW // CCH


@functools.lru_cache(maxsize=None)
def _make_sc_combine():
    mesh = plsc.VectorSubcoreMesh(core_axis_name="c", subcore_axis_name="s",
                                  num_cores=NC, num_subcores=NS)

    @functools.partial(
        pl.kernel,
        out_type=jax.ShapeDtypeStruct((T, H), jnp.float32),
        mesh=mesh,
        scratch_types=[
            pltpu.VMEM((CNCH, CCH), jnp.int32),
            pltpu.VMEM((CNCH, CCH), jnp.int32),
            pltpu.VMEM((CCH, H), jnp.float32),
            pltpu.VMEM((CCH, H), jnp.float32),
            pltpu.VMEM((CCH, H), jnp.float32),
            pltpu.VMEM((CCH, H), jnp.float32),
            pltpu.VMEM((CCH, H), jnp.float32),
            pltpu.VMEM((CCH, H), jnp.float32),
            pltpu.SemaphoreType.DMA,
            pltpu.SemaphoreType.DMA,
            pltpu.SemaphoreType.DMA,
            pltpu.SemaphoreType.DMA,
        ],
    )
    def sc_combine(ys_hbm, sa_hbm, sb_hbm, out_hbm,
                   ia_v, ib_v, a0, a1, b0, b1, o0, o1, sg0, sg1, so0, so1):
        wid = lax.axis_index("s") * NC + lax.axis_index("c")
        pltpu.sync_copy(sa_hbm.at[wid], ia_v)
        pltpu.sync_copy(sb_hbm.at[wid], ib_v)
        a = (a0, a1)
        b = (b0, b1)
        o = (o0, o1)
        sg = (sg0, sg1)
        so = (so0, so1)
        ga = [None, None]
        gb = [None, None]
        oc = [None, None]
        ga[0] = pltpu.async_copy(ys_hbm.at[ia_v.at[0]], a0, sg0)
        gb[0] = pltpu.async_copy(ys_hbm.at[ib_v.at[0]], b0, sg0)
        for c in range(CNCH):
            p = c % 2
            ga[p].wait()
            gb[p].wait()
            if c + 1 < CNCH:
                q = (c + 1) % 2
                ga[q] = pltpu.async_copy(ys_hbm.at[ia_v.at[c + 1]], a[q], sg[q])
                gb[q] = pltpu.async_copy(ys_hbm.at[ib_v.at[c + 1]], b[q], sg[q])
            if c >= 2:
                oc[p].wait()
            av, bv, ov = a[p], b[p], o[p]

            def row_add(r, carry, av=av, bv=bv, ov=ov):
                for u in range(H // 16):
                    s = pl.ds(u * 16, 16)
                    ov[r, s] = av[r, s] + bv[r, s]
                return carry

            lax.fori_loop(0, CCH, row_add, 0)
            oc[p] = pltpu.async_copy(
                ov, out_hbm.at[pl.ds(wid * TPW + c * CCH, CCH)], so[p])
        oc[0].wait()
        oc[1].wait()

    return sc_combine


def _sc_combine(ys, slots_a, slots_b):
    return _make_sc_combine()(
        ys, slots_a.reshape(NW, CNCH, CCH), slots_b.reshape(NW, CNCH, CCH))


def kernel(hidden_states, gate_proj_w, gate_weights, up_weights, down_weights):
    dests, w2, xb, meta_row = _router(hidden_states, gate_proj_w)
    meta = meta_row.reshape(128)
    ys = _ffn(meta, xb, dests, w2, gate_weights, up_weights, down_weights)
    return _sc_combine(ys, dests[0, :], dests[1, :])


# speedup vs baseline: 1.4156x; 1.4156x over previous
import functools

import jax
import jax.numpy as jnp
from jax import lax
from jax.experimental import pallas as pl
from jax.experimental.pallas import tpu as pltpu
from jax.experimental.pallas import tpu_sc as plsc

T = 2048
H = 1024
I = 1024
E = 8
K = 2
P = T * K
TILE = 256
NT = 24
NP = NT * TILE

NC, NS = 2, 16
NW = NC * NS
RPW = NP // NW
GCH = 64
TPW = T // NW
CCH = 16


def _router_kernel(x_ref, gw_ref, dest_ref, w_ref, xi_ref, meta_ref):
    xi_ref[...] = x_ref[...].astype(jnp.bfloat16)
    logits = lax.dot_general(
        x_ref[...], gw_ref[...], (((1,), (1,)), ((), ())),
        preferred_element_type=jnp.float32)
    a1 = jnp.argmax(logits, axis=1)
    l1 = jnp.max(logits, axis=1)
    cols = lax.broadcasted_iota(jnp.int32, (T, E), 1)
    masked = jnp.where(cols == a1[:, None], -jnp.inf, logits)
    a2 = jnp.argmax(masked, axis=1)
    l2 = jnp.max(masked, axis=1)
    w1 = jax.nn.sigmoid(l1 - l2)
    w_ref[0, :] = w1
    w_ref[1, :] = 1.0 - w1

    onea = (cols == a1[:, None]).astype(jnp.float32)
    oneb = (cols == a2[:, None]).astype(jnp.float32)
    s = onea + oneb
    cum = s
    for step in (1, 2, 4, 8, 16, 32, 64, 128, 256, 512, 1024):
        cum = cum + jnp.concatenate(
            [jnp.zeros((step, E), jnp.float32), cum[:T - step]], axis=0)
    cum_excl = cum - s
    counts = cum[T - 1, :].reshape(1, E)
    pad_counts = jnp.floor((counts + (TILE - 1)) * (1.0 / TILE)) * TILE
    rr_r = lax.broadcasted_iota(jnp.int32, (E, E), 0)
    rr_c = lax.broadcasted_iota(jnp.int32, (E, E), 1)
    tri_x = (rr_c > rr_r).astype(jnp.float32)
    pad_off = jnp.dot(pad_counts, tri_x,
                      preferred_element_type=jnp.float32)
    rank0 = jnp.sum(onea * cum_excl, axis=1)
    rank1 = jnp.sum(oneb * cum_excl, axis=1)
    off0 = jnp.sum(onea * pad_off, axis=1)
    off1 = jnp.sum(oneb * pad_off, axis=1)
    dest_ref[0, :] = (off0 + rank0).astype(jnp.int32)
    dest_ref[1, :] = (off1 + rank1).astype(jnp.int32)

    pad_end = pad_off + pad_counts
    n_tiles_f = pad_end[0, E - 1] * (1.0 / TILE)
    i128 = lax.broadcasted_iota(jnp.int32, (128,), 0).astype(jnp.float32)
    g128 = i128 * float(TILE)
    te = jnp.minimum(
        jnp.sum((g128[:, None] >= pad_end).astype(jnp.float32), axis=1),
        float(E - 1))
    last_e = jnp.sum(jnp.where(i128 == n_tiles_f - 1.0, te, 0.0))
    te = jnp.where(i128 < n_tiles_f, te, last_e)
    meta = jnp.where(i128 == float(NT), n_tiles_f, te)
    meta_ref[0, :] = meta.astype(jnp.int32)


def _router(x, gate_proj_w):
    return pl.pallas_call(
        _router_kernel,
        in_specs=[
            pl.BlockSpec((T, H), lambda: (0, 0)),
            pl.BlockSpec((E, H), lambda: (0, 0)),
        ],
        out_specs=[
            pl.BlockSpec((K, T), lambda: (0, 0)),
            pl.BlockSpec((K, T), lambda: (0, 0)),
            pl.BlockSpec((T, H), lambda: (0, 0)),
            pl.BlockSpec((1, 128), lambda: (0, 0)),
        ],
        out_shape=[
            jax.ShapeDtypeStruct((K, T), jnp.int32),
            jax.ShapeDtypeStruct((K, T), jnp.float32),
            jax.ShapeDtypeStruct((T, H), jnp.bfloat16),
            jax.ShapeDtypeStruct((1, 128), jnp.int32),
        ],
    )(x, gate_proj_w)


def _ffn_kernel(meta_ref, xb_ref, dest_ref, w2_ref, wg_ref, wu_ref, wd_ref,
                ys_ref):
    g = pl.program_id(0)

    @pl.when(g < meta_ref[NT])
    def _():
        rows = lax.broadcasted_iota(jnp.int32, (TILE, T), 0) + g * TILE
        m0 = (dest_ref[0, :][None, :] == rows).astype(jnp.bfloat16)
        m1 = (dest_ref[1, :][None, :] == rows).astype(jnp.bfloat16)
        ms = m0 + m1
        xg = jnp.dot(ms, xb_ref[...],
                     preferred_element_type=jnp.float32).astype(jnp.bfloat16)
        w = (jnp.dot(m0, w2_ref[0, :].astype(jnp.bfloat16),
                     preferred_element_type=jnp.float32)
             + jnp.dot(m1, w2_ref[1, :].astype(jnp.bfloat16),
                       preferred_element_type=jnp.float32))
        wg = wg_ref[0].astype(jnp.bfloat16)
        wu = wu_ref[0].astype(jnp.bfloat16)
        wd = wd_ref[0].astype(jnp.bfloat16)
        gate = jnp.dot(xg, wg, preferred_element_type=jnp.float32)
        up = jnp.dot(xg, wu, preferred_element_type=jnp.float32)
        inter = (gate * jax.nn.sigmoid(gate) * up).astype(jnp.bfloat16)
        d = jnp.dot(inter, wd, preferred_element_type=jnp.float32)
        ys_ref[...] = w[:, None] * d


def _ffn(meta, xb, dests, w2, gate_weights, up_weights, down_weights):
    grid_spec = pltpu.PrefetchScalarGridSpec(
        num_scalar_prefetch=1,
        grid=(NT,),
        in_specs=[
            pl.BlockSpec((T, H), lambda g, m: (0, 0)),
            pl.BlockSpec((K, T), lambda g, m: (0, 0)),
            pl.BlockSpec((K, T), lambda g, m: (0, 0)),
            pl.BlockSpec((1, H, I), lambda g, m: (m[g], 0, 0)),
            pl.BlockSpec((1, H, I), lambda g, m: (m[g], 0, 0)),
            pl.BlockSpec((1, I, H), lambda g, m: (m[g], 0, 0)),
        ],
        out_specs=pl.BlockSpec((TILE, H), lambda g, m: (g, 0)),
    )
    return pl.pallas_call(
        _ffn_kernel,
        grid_spec=grid_spec,
        out_shape=jax.ShapeDtypeStruct((NP, H), jnp.float32),
    )(meta, xb, dests, w2, gate_weights, up_weights, down_weights)


CNCH = TPW // CCH


@functools.lru_cache(maxsize=None)
def _make_sc_combine():
    mesh = plsc.VectorSubcoreMesh(core_axis_name="c", subcore_axis_name="s",
                                  num_cores=NC, num_subcores=NS)

    @functools.partial(
        pl.kernel,
        out_type=jax.ShapeDtypeStruct((T, H), jnp.float32),
        mesh=mesh,
        scratch_types=[
            pltpu.VMEM((CNCH, CCH), jnp.int32),
            pltpu.VMEM((CNCH, CCH), jnp.int32),
            pltpu.VMEM((CCH, H), jnp.float32),
            pltpu.VMEM((CCH, H), jnp.float32),
            pltpu.VMEM((CCH, H), jnp.float32),
            pltpu.VMEM((CCH, H), jnp.float32),
            pltpu.VMEM((CCH, H), jnp.float32),
            pltpu.VMEM((CCH, H), jnp.float32),
            pltpu.SemaphoreType.DMA,
            pltpu.SemaphoreType.DMA,
            pltpu.SemaphoreType.DMA,
            pltpu.SemaphoreType.DMA,
        ],
    )
    def sc_combine(ys_hbm, sa_hbm, sb_hbm, out_hbm,
                   ia_v, ib_v, a0, a1, b0, b1, o0, o1, sg0, sg1, so0, so1):
        wid = lax.axis_index("s") * NC + lax.axis_index("c")
        pltpu.sync_copy(sa_hbm.at[wid], ia_v)
        pltpu.sync_copy(sb_hbm.at[wid], ib_v)
        a = (a0, a1)
        b = (b0, b1)
        o = (o0, o1)
        sg = (sg0, sg1)
        so = (so0, so1)
        ga = [None, None]
        gb = [None, None]
        oc = [None, None]
        ga[0] = pltpu.async_copy(ys_hbm.at[ia_v.at[0]], a0, sg0)
        gb[0] = pltpu.async_copy(ys_hbm.at[ib_v.at[0]], b0, sg0)
        for c in range(CNCH):
            p = c % 2
            ga[p].wait()
            gb[p].wait()
            if c + 1 < CNCH:
                q = (c + 1) % 2
                ga[q] = pltpu.async_copy(ys_hbm.at[ia_v.at[c + 1]], a[q], sg[q])
                gb[q] = pltpu.async_copy(ys_hbm.at[ib_v.at[c + 1]], b[q], sg[q])
            if c >= 2:
                oc[p].wait()
            av, bv, ov = a[p], b[p], o[p]

            def row_add(r, carry, av=av, bv=bv, ov=ov):
                for u in range(H // 16):
                    s = pl.ds(u * 16, 16)
                    ov[r, s] = av[r, s] + bv[r, s]
                return carry

            lax.fori_loop(0, CCH, row_add, 0)
            oc[p] = pltpu.async_copy(
                ov, out_hbm.at[pl.ds(wid * TPW + c * CCH, CCH)], so[p])
        oc[0].wait()
        oc[1].wait()

    return sc_combine


def _sc_combine(ys, slots_a, slots_b):
    return _make_sc_combine()(
        ys, slots_a.reshape(NW, CNCH, CCH), slots_b.reshape(NW, CNCH, CCH))


def kernel(hidden_states, gate_proj_w, gate_weights, up_weights, down_weights):
    dests, w2, xb, meta_row = _router(hidden_states, gate_proj_w)
    meta = meta_row.reshape(128)
    ys = _ffn(meta, xb, dests, w2, gate_weights, up_weights, down_weights)
    return _sc_combine(ys, dests[0, :], dests[1, :])
